# baseline (speedup 1.0000x reference)
import numpy as np
import jax
import jax.numpy as jnp
from jax.experimental import pallas as pl
from jax.experimental.pallas import tpu as pltpu

B, T, D, S, E, H = 2, 2048, 1024, 8, 8, 16
KT, KR, KW = 2, 4, 2
DH = D // H
NEG = -1e30
TK = 256
NT = T // TK
TA = D // NT
_C1 = (0.25 / KT) ** 2


def _pe_np():
    pos = np.arange(T, dtype=np.float64)[:, None]
    half = D // 2
    freqs = float(S) ** (np.arange(half, dtype=np.float64) / half)
    return np.concatenate([np.sin(pos / freqs), np.cos(pos / freqs)],
                          axis=-1).astype(np.float32)


_PE_BF = _pe_np().astype(jnp.bfloat16)


def _pe_last_np():
    pos = float(T - 1)
    half = D // 2
    freqs = float(S) ** (np.arange(half, dtype=np.float64) / half)
    return np.concatenate([np.sin(pos / freqs),
                           np.cos(pos / freqs)]).astype(np.float32)[None, :]


_PE_LAST = _pe_last_np()


def _ln_rows(x, g, b, eps=1e-5):
    m = jnp.mean(x, axis=1, keepdims=True)
    v = jnp.mean((x - m) ** 2, axis=1, keepdims=True)
    return (x - m) * jax.lax.rsqrt(v + eps) * g + b


def _iota(shape, dim):
    return jax.lax.broadcasted_iota(jnp.int32, shape, dim)


def _routing_compute(state, rgw, lnmg, lnmb, gatew, lnsg, lnsb, scw,
                     d2sum, read_idx_ref, experts_ref, widx_ref):
    scores16 = jax.lax.dot_general(
        rgw, state, (((1,), (1,)), ((), ())),
        preferred_element_type=jnp.float32)
    col = _iota((B, B * S), 1)
    brow = _iota((B, B * S), 0)
    gmask = (col // S) == brow
    smask = jnp.where(gmask, scores16, NEG)
    iota16 = col.astype(jnp.float32)
    onehot_cols4 = _iota((B, KR), 1)
    bcol = _iota((B, 1), 0).astype(jnp.float32) * float(S)

    P = jnp.zeros((B * KR, B * S), jnp.float32)
    read_idx_f = jnp.zeros((B, KR), jnp.float32)
    eqs = []
    rr = _iota((B * KR, B), 0)
    bb = _iota((B * KR, B), 1)
    for j in range(KR):
        m = jnp.max(smask, axis=1, keepdims=True)
        cand = jnp.where(smask == m, iota16, 1e9)
        idxf = jnp.min(cand, axis=1, keepdims=True)
        eq = (iota16 == idxf)
        eqs.append(eq)
        smask = jnp.where(eq, NEG, smask)
        indj = jnp.where((rr // KR == bb) & (rr % KR == j), 1.0, 0.0)
        P = P + jnp.dot(indj, eq.astype(jnp.float32),
                        preferred_element_type=jnp.float32)
        read_idx_f = read_idx_f + (idxf - bcol) * (onehot_cols4 == j)
    read_idx_ref[:] = read_idx_f.astype(jnp.int32)

    latent_read = jnp.dot(P, state, preferred_element_type=jnp.float32)
    lm = _ln_rows(latent_read, lnmg, lnmb)
    gl = jnp.dot(lm, gatew, preferred_element_type=jnp.float32)
    iota8 = _iota((B * KR, E), 1).astype(jnp.float32)
    for l in range(KT):
        m = jnp.max(gl, axis=1, keepdims=True)
        idxf = jnp.min(jnp.where(gl == m, iota8, 1e9), axis=1, keepdims=True)
        experts_ref[:, l:l + 1] = idxf.astype(jnp.int32)
        gl = jnp.where(iota8 == idxf, NEG, gl)

    diagQ = 1.0 - 2.0 * _C1 * d2sum
    th = jnp.tanh(latent_read * diagQ)
    gavg = jnp.where(_iota((B, B * KR), 1) // KR == _iota((B, B * KR), 0),
                     1.0 / KR, 0.0)
    lca = jnp.dot(gavg, th, preferred_element_type=jnp.float32)
    tl = jnp.tanh(lca)

    ls = _ln_rows(state, lnsg, lnsb)
    slot_row = jax.lax.dot_general(
        scw, ls, (((1,), (1,)), ((), ())),
        preferred_element_type=jnp.float32)
    logits_read = jnp.zeros((B, KR), jnp.float32)
    for j in range(KR):
        lrj = jnp.sum(jnp.where(eqs[j], slot_row, 0.0), axis=1,
                      keepdims=True)
        logits_read = logits_read + lrj * (onehot_cols4 == j)

    lr = logits_read
    iota4 = _iota((B, KR), 1).astype(jnp.float32)
    wsel = []
    for l in range(KW):
        m = jnp.max(lr, axis=1, keepdims=True)
        idlf = jnp.min(jnp.where(lr == m, iota4, 1e9), axis=1, keepdims=True)
        sel = (iota4 == idlf)
        wloc = jnp.sum(jnp.where(sel, read_idx_f, 0.0), axis=1,
                       keepdims=True)
        widx_ref[:, l:l + 1] = wloc.astype(jnp.int32)
        wsel.append(wloc)
        lr = jnp.where(sel, NEG, lr)

    l2 = logits_read[:, 0:KW]
    mm = jnp.max(l2, axis=1, keepdims=True)
    ee = jnp.exp(l2 - mm)
    w_soft = ee / jnp.sum(ee, axis=1, keepdims=True)

    iota8r = _iota((B, S), 1).astype(jnp.float32)
    alpha = jnp.zeros((B, S), jnp.float32)
    for l in range(KW):
        alpha = alpha + jnp.where(iota8r == wsel[l],
                                  w_soft[:, l:l + 1], 0.0)
    asum = jnp.sum(alpha, axis=1, keepdims=True)

    Ec = jnp.where(_iota((B * S, B), 0) // S == _iota((B * S, B), 1),
                   1.0, 0.0)
    tlx = jnp.dot(Ec, tl, preferred_element_type=jnp.float32)
    tmp = jnp.dot(Ec, alpha, preferred_element_type=jnp.float32)
    scmask = (_iota((B * S, S), 0) % S) == _iota((B * S, S), 1)
    alpha_col = jnp.sum(jnp.where(scmask, tmp, 0.0), axis=1, keepdims=True)
    asx = jnp.dot(Ec, asum, preferred_element_type=jnp.float32)
    return alpha_col * tlx + (1.0 - asx) * state


def _mega_kernel(x_ref, pe_ref, A_ref, Woe_ref, boe_ref, lnkg_ref, lnkb_ref,
                 xlast_ref, pelast_ref, lnqg_ref, lnqb_ref, Wq_ref,
                 bqcol_ref, Wk_ref,
                 state_in_ref, rgw_ref, lnmg_ref, lnmb_ref, gatew_ref,
                 lnsg_ref, lnsb_ref, scw_ref,
                 Wqkv_hbm, Wo_hbm, W1_hbm, W2_hbm, Wop_hbm,
                 bv_ref, bo_ref, lnfg_ref, lnfb_ref, b1_ref, b2_ref,
                 bop_ref,
                 y_ref, read_idx_ref, experts_ref, widx_ref, state_out_ref,
                 qk0_sc, qk1_sc, lolast_sc, m_sc, l_sc, acc_sc, d2_sc,
                 state_sc, wv_sc, wo_sc, w1_sc, w2_sc, wop_sc,
                 sem_v, sem_o, sem_1, sem_2, sem_p):
    t = pl.program_id(0)
    b = pl.program_id(1)

    @pl.when((t == 0) & (b == 0))
    def _prologue():
        pltpu.make_async_copy(Wqkv_hbm.at[:, pl.ds(2 * D, D)],
                              wv_sc, sem_v).start()
        pltpu.make_async_copy(Wo_hbm, wo_sc, sem_o).start()
        pltpu.make_async_copy(W1_hbm, w1_sc, sem_1).start()
        pltpu.make_async_copy(W2_hbm, w2_sc, sem_2).start()
        pltpu.make_async_copy(Wop_hbm, wop_sc, sem_p).start()
        d2_sc[:] = jnp.zeros((1, D), jnp.float32)
        lo = jnp.dot(xlast_ref[:], Woe_ref[:],
                     preferred_element_type=jnp.float32)
        lo = lo + boe_ref[:] + pelast_ref[:]
        lolast_sc[:] = lo
        qn = _ln_rows(lo, lnqg_ref[:], lnqb_ref[:])
        qcol = jax.lax.dot_general(
            Wq_ref[:], qn, (((0,), (1,)), ((), ())),
            preferred_element_type=jnp.float32) + bqcol_ref[:]
        di = _iota((D, H), 0)
        hi = _iota((D, H), 1)
        qk0_sc[:] = jnp.dot(
            Wk_ref[:], jnp.where(di // DH == hi, qcol[:, 0:1], 0.0),
            preferred_element_type=jnp.float32)
        qk1_sc[:] = jnp.dot(
            Wk_ref[:], jnp.where(di // DH == hi, qcol[:, 1:2], 0.0),
            preferred_element_type=jnp.float32)

    @pl.when(t == 0)
    def _init_flash():
        m_sc[pl.ds(b, 1), :] = jnp.full((1, H), NEG, jnp.float32)
        l_sc[pl.ds(b, 1), :] = jnp.zeros((1, H), jnp.float32)
        acc_sc[pl.ds(b * H, H), :] = jnp.zeros((H, D), jnp.float32)

    @pl.when(b == 0)
    def _d2_accum():
        a = A_ref[:]
        d2_sc[:] = d2_sc[:] + jnp.sum(a * a, axis=0, keepdims=True)

    QK = jnp.where(b == 0, qk0_sc[:], qk1_sc[:])
    hm = jnp.where(_iota((H, D), 1) // DH == _iota((H, D), 0),
                   1.0, 0.0)

    def update(Sg, rows):
        m_old = m_sc[pl.ds(b, 1), :]
        m_new = jnp.maximum(m_old, jnp.max(Sg, axis=0, keepdims=True))
        corr = jnp.exp(m_old - m_new)
        Eu = jnp.exp(Sg - m_new)
        l_sc[pl.ds(b, 1), :] = l_sc[pl.ds(b, 1), :] * corr + jnp.sum(
            Eu, axis=0, keepdims=True)
        corr_f = jnp.dot(corr, hm, preferred_element_type=jnp.float32)
        corr_col = jax.lax.dot_general(
            hm, corr_f, (((1,), (1,)), ((), ())),
            preferred_element_type=jnp.float32) * (1.0 / DH)
        wsum = jax.lax.dot_general(
            Eu, rows, (((0,), (0,)), ((), ())),
            preferred_element_type=jnp.float32)
        acc_sc[pl.ds(b * H, H), :] = (
            acc_sc[pl.ds(b * H, H), :] * corr_col + wsum)
        m_sc[pl.ds(b, 1), :] = m_new

    x = x_ref[0]
    pe = pe_ref[:].astype(jnp.float32)
    lo = jnp.dot(x, Woe_ref[:],
                 preferred_element_type=jnp.float32) + boe_ref[:] + pe
    kvn = _ln_rows(lo, lnkg_ref[:], lnkb_ref[:])
    Sg = jnp.dot(kvn, QK, preferred_element_type=jnp.float32) * (1.0 / 8.0)
    update(Sg, kvn)

    @pl.when((t == NT - 1) & (b == 0))
    def _routing():
        state_new = _routing_compute(
            state_in_ref[:], rgw_ref[:], lnmg_ref[:], lnmb_ref[:],
            gatew_ref[:], lnsg_ref[:], lnsb_ref[:], scw_ref[:],
            d2_sc[:], read_idx_ref, experts_ref, widx_ref)
        state_sc[:] = state_new
        state_out_ref[:] = state_new

    @pl.when(t == NT - 1)
    def _state_rows():
        st = state_sc[pl.ds(b * S, S), :]
        kvn8 = _ln_rows(st, lnkg_ref[:], lnkb_ref[:])
        S8 = jnp.dot(kvn8, QK, preferred_element_type=jnp.float32) * (1.0 / 8.0)
        update(S8, kvn8)

    @pl.when((t == NT - 1) & (b == B - 1))
    def _epilogue():
        pltpu.make_async_copy(Wqkv_hbm.at[:, pl.ds(2 * D, D)],
                              wv_sc, sem_v).wait()
        pltpu.make_async_copy(Wo_hbm, wo_sc, sem_o).wait()
        pltpu.make_async_copy(W1_hbm, w1_sc, sem_1).wait()
        pltpu.make_async_copy(W2_hbm, w2_sc, sem_2).wait()
        pltpu.make_async_copy(Wop_hbm, wop_sc, sem_p).wait()
        full = jnp.dot(acc_sc[:], wv_sc[:],
                       preferred_element_type=jnp.float32)
        rows = []
        for bb in range(B):
            fb = full[bb * H:(bb + 1) * H]
            lfull = jnp.dot(l_sc[bb:bb + 1], hm,
                            preferred_element_type=jnp.float32)
            rows.append(jnp.sum(jnp.where(hm > 0.0, fb, 0.0), axis=0,
                                keepdims=True) / lfull + bv_ref[:])
        attn = jnp.concatenate(rows, axis=0)
        ao = jnp.dot(attn, wo_sc[:],
                     preferred_element_type=jnp.float32) + bo_ref[:]
        ll = lolast_sc[:] + ao
        hdn = _ln_rows(ll, lnfg_ref[:], lnfb_ref[:])
        h1 = jnp.dot(hdn, w1_sc[:],
                     preferred_element_type=jnp.float32) + b1_ref[:]
        g = 0.5 * h1 * (1.0 + jax.lax.erf(h1 * (0.5 ** 0.5)))
        ll2 = ll + jnp.dot(g, w2_sc[:],
                           preferred_element_type=jnp.float32) + b2_ref[:]
        y_ref[:] = jnp.dot(ll2, wop_sc[:],
                           preferred_element_type=jnp.float32) + bop_ref[:]


def kernel(x, state_flat, read_gate_w, ln_moe_g, ln_moe_b, gate_w, A_skew,
           ln_slot_g, ln_slot_b, slot_ctx_w, out_emb_W, out_emb_b,
           ln_q_g, ln_q_b, ln_kv_g, ln_kv_b, mha_Wqkv, mha_bqkv,
           mha_Wo, mha_bo, ln_ffn_g, ln_ffn_b, ffn_W1, ffn_b1,
           ffn_W2, ffn_b2, out_proj_W, out_proj_b):
    f32 = jnp.float32
    state16 = state_flat.reshape(B * S, D)
    row = lambda v: v.reshape(1, D)
    bq = mha_bqkv[:D]
    bv = mha_bqkv[2 * D:]

    cst = lambda shape: pl.BlockSpec(shape, lambda t, b: tuple(
        0 for _ in shape))
    hbm = pl.BlockSpec(memory_space=pltpu.MemorySpace.HBM)

    y, read_idx, experts, write_idx, state_new = pl.pallas_call(
        _mega_kernel,
        grid=(NT, B),
        in_specs=[
            pl.BlockSpec((1, TK, D), lambda t, b: (b, t, 0)),
            pl.BlockSpec((TK, D), lambda t, b: (t, 0)),
            pl.BlockSpec((TA, D), lambda t, b: (t, 0)),
            cst((D, D)),
            cst((1, D)),
            cst((1, D)), cst((1, D)),
            cst((B, D)),
            cst((1, D)),
            cst((1, D)), cst((1, D)),
            pl.BlockSpec((D, D), lambda t, b: (0, 0)),
            cst((D, 1)),
            pl.BlockSpec((D, D), lambda t, b: (0, 1)),
            cst((B * S, D)),
            cst((1, D)),
            cst((1, D)), cst((1, D)),
            cst((D, E)),
            cst((1, D)), cst((1, D)),
            cst((1, D)),
            hbm, hbm, hbm, hbm, hbm,
            cst((1, D)), cst((1, D)),
            cst((1, D)), cst((1, D)),
            cst((1, D)), cst((1, D)),
            cst((1, D)),
        ],
        out_specs=(
            cst((B, D)),
            cst((B, KR)),
            cst((B * KR, KT)),
            cst((B, KW)),
            cst((B * S, D)),
        ),
        out_shape=(
            jax.ShapeDtypeStruct((B, D), f32),
            jax.ShapeDtypeStruct((B, KR), jnp.int32),
            jax.ShapeDtypeStruct((B * KR, KT), jnp.int32),
            jax.ShapeDtypeStruct((B, KW), jnp.int32),
            jax.ShapeDtypeStruct((B * S, D), f32),
        ),
        scratch_shapes=[
            pltpu.VMEM((D, H), f32), pltpu.VMEM((D, H), f32),
            pltpu.VMEM((B, D), f32),
            pltpu.VMEM((B, H), f32), pltpu.VMEM((B, H), f32),
            pltpu.VMEM((B * H, D), f32),
            pltpu.VMEM((1, D), f32),
            pltpu.VMEM((B * S, D), f32),
            pltpu.VMEM((D, D), f32), pltpu.VMEM((D, D), f32),
            pltpu.VMEM((D, D), f32), pltpu.VMEM((D, D), f32),
            pltpu.VMEM((D, D), f32),
            pltpu.SemaphoreType.DMA, pltpu.SemaphoreType.DMA,
            pltpu.SemaphoreType.DMA, pltpu.SemaphoreType.DMA,
            pltpu.SemaphoreType.DMA,
        ],
    )(x, _PE_BF, A_skew, out_emb_W, row(out_emb_b), row(ln_kv_g),
      row(ln_kv_b),
      x[:, -1, :], _PE_LAST, row(ln_q_g), row(ln_q_b), mha_Wqkv,
      bq.reshape(D, 1), mha_Wqkv,
      state16, row(read_gate_w), row(ln_moe_g), row(ln_moe_b), gate_w,
      row(ln_slot_g), row(ln_slot_b), row(slot_ctx_w),
      mha_Wqkv, mha_Wo, ffn_W1, ffn_W2, out_proj_W,
      bv.reshape(1, D), row(mha_bo), row(ln_ffn_g), row(ln_ffn_b),
      row(ffn_b1), row(ffn_b2), row(out_proj_b))

    return (y, experts.reshape(B, KR, KT), read_idx, write_idx,
            state_new.reshape(B, S * D))


# speedup vs baseline: 1.2066x; 1.2066x over previous
import numpy as np
import jax
import jax.numpy as jnp
from jax.experimental import pallas as pl
from jax.experimental.pallas import tpu as pltpu

B, T, D, S, E, H = 2, 2048, 1024, 8, 8, 16
KT, KR, KW = 2, 4, 2
DH = D // H
NEG = -1e30
TK = 1024
NT = T // TK
TA = D // NT
_C1 = (0.25 / KT) ** 2


def _pe_np():
    pos = np.arange(T, dtype=np.float64)[:, None]
    half = D // 2
    freqs = float(S) ** (np.arange(half, dtype=np.float64) / half)
    return np.concatenate([np.sin(pos / freqs), np.cos(pos / freqs)],
                          axis=-1).astype(np.float32)


_PE_BF = _pe_np().astype(jnp.bfloat16)


def _pe_last_np():
    pos = float(T - 1)
    half = D // 2
    freqs = float(S) ** (np.arange(half, dtype=np.float64) / half)
    return np.concatenate([np.sin(pos / freqs),
                           np.cos(pos / freqs)]).astype(np.float32)[None, :]


_PE_LAST = _pe_last_np()


def _ln_rows(x, g, b, eps=1e-5):
    m = jnp.mean(x, axis=1, keepdims=True)
    v = jnp.mean((x - m) ** 2, axis=1, keepdims=True)
    return (x - m) * jax.lax.rsqrt(v + eps) * g + b


def _iota(shape, dim):
    return jax.lax.broadcasted_iota(jnp.int32, shape, dim)


def _routing_compute(state, rgw, lnmg, lnmb, gatew, lnsg, lnsb, scw,
                     d2sum, read_idx_ref, experts_ref, widx_ref):
    scores16 = jax.lax.dot_general(
        rgw, state, (((1,), (1,)), ((), ())),
        preferred_element_type=jnp.float32)
    col = _iota((B, B * S), 1)
    brow = _iota((B, B * S), 0)
    gmask = (col // S) == brow
    smask = jnp.where(gmask, scores16, NEG)
    iota16 = col.astype(jnp.float32)
    onehot_cols4 = _iota((B, KR), 1)
    bcol = _iota((B, 1), 0).astype(jnp.float32) * float(S)

    P = jnp.zeros((B * KR, B * S), jnp.float32)
    read_idx_f = jnp.zeros((B, KR), jnp.float32)
    eqs = []
    rr = _iota((B * KR, B), 0)
    bb = _iota((B * KR, B), 1)
    for j in range(KR):
        m = jnp.max(smask, axis=1, keepdims=True)
        cand = jnp.where(smask == m, iota16, 1e9)
        idxf = jnp.min(cand, axis=1, keepdims=True)
        eq = (iota16 == idxf)
        eqs.append(eq)
        smask = jnp.where(eq, NEG, smask)
        indj = jnp.where((rr // KR == bb) & (rr % KR == j), 1.0, 0.0)
        P = P + jnp.dot(indj, eq.astype(jnp.float32),
                        preferred_element_type=jnp.float32)
        read_idx_f = read_idx_f + (idxf - bcol) * (onehot_cols4 == j)
    read_idx_ref[:] = read_idx_f.astype(jnp.int32)

    latent_read = jnp.dot(P, state, preferred_element_type=jnp.float32)
    lm = _ln_rows(latent_read, lnmg, lnmb)
    gl = jnp.dot(lm, gatew, preferred_element_type=jnp.float32)
    iota8 = _iota((B * KR, E), 1).astype(jnp.float32)
    for l in range(KT):
        m = jnp.max(gl, axis=1, keepdims=True)
        idxf = jnp.min(jnp.where(gl == m, iota8, 1e9), axis=1, keepdims=True)
        experts_ref[:, l:l + 1] = idxf.astype(jnp.int32)
        gl = jnp.where(iota8 == idxf, NEG, gl)

    diagQ = 1.0 - 2.0 * _C1 * d2sum
    th = jnp.tanh(latent_read * diagQ)
    gavg = jnp.where(_iota((B, B * KR), 1) // KR == _iota((B, B * KR), 0),
                     1.0 / KR, 0.0)
    lca = jnp.dot(gavg, th, preferred_element_type=jnp.float32)
    tl = jnp.tanh(lca)

    ls = _ln_rows(state, lnsg, lnsb)
    slot_row = jax.lax.dot_general(
        scw, ls, (((1,), (1,)), ((), ())),
        preferred_element_type=jnp.float32)
    logits_read = jnp.zeros((B, KR), jnp.float32)
    for j in range(KR):
        lrj = jnp.sum(jnp.where(eqs[j], slot_row, 0.0), axis=1,
                      keepdims=True)
        logits_read = logits_read + lrj * (onehot_cols4 == j)

    lr = logits_read
    iota4 = _iota((B, KR), 1).astype(jnp.float32)
    wsel = []
    for l in range(KW):
        m = jnp.max(lr, axis=1, keepdims=True)
        idlf = jnp.min(jnp.where(lr == m, iota4, 1e9), axis=1, keepdims=True)
        sel = (iota4 == idlf)
        wloc = jnp.sum(jnp.where(sel, read_idx_f, 0.0), axis=1,
                       keepdims=True)
        widx_ref[:, l:l + 1] = wloc.astype(jnp.int32)
        wsel.append(wloc)
        lr = jnp.where(sel, NEG, lr)

    l2 = logits_read[:, 0:KW]
    mm = jnp.max(l2, axis=1, keepdims=True)
    ee = jnp.exp(l2 - mm)
    w_soft = ee / jnp.sum(ee, axis=1, keepdims=True)

    iota8r = _iota((B, S), 1).astype(jnp.float32)
    alpha = jnp.zeros((B, S), jnp.float32)
    for l in range(KW):
        alpha = alpha + jnp.where(iota8r == wsel[l],
                                  w_soft[:, l:l + 1], 0.0)
    asum = jnp.sum(alpha, axis=1, keepdims=True)

    Ec = jnp.where(_iota((B * S, B), 0) // S == _iota((B * S, B), 1),
                   1.0, 0.0)
    tlx = jnp.dot(Ec, tl, preferred_element_type=jnp.float32)
    tmp = jnp.dot(Ec, alpha, preferred_element_type=jnp.float32)
    scmask = (_iota((B * S, S), 0) % S) == _iota((B * S, S), 1)
    alpha_col = jnp.sum(jnp.where(scmask, tmp, 0.0), axis=1, keepdims=True)
    asx = jnp.dot(Ec, asum, preferred_element_type=jnp.float32)
    return alpha_col * tlx + (1.0 - asx) * state


def _mega_kernel(x_ref, pe_ref, A_ref, Woe_ref, boe_ref, lnkg_ref, lnkb_ref,
                 xlast_ref, pelast_ref, lnqg_ref, lnqb_ref, Wq_ref,
                 bqcol_ref, Wk_ref,
                 state_in_ref, rgw_ref, lnmg_ref, lnmb_ref, gatew_ref,
                 lnsg_ref, lnsb_ref, scw_ref,
                 Wqkv_hbm, Wo_hbm, W1_hbm, W2_hbm, Wop_hbm,
                 bv_ref, bo_ref, lnfg_ref, lnfb_ref, b1_ref, b2_ref,
                 bop_ref,
                 y_ref, read_idx_ref, experts_ref, widx_ref, state_out_ref,
                 qk0_sc, qk1_sc, lolast_sc, m_sc, l_sc, acc_sc, d2_sc,
                 state_sc, wv_sc, wo_sc, w1_sc, w2_sc, wop_sc,
                 sem_v, sem_o, sem_1, sem_2, sem_p):
    t = pl.program_id(0)
    b = pl.program_id(1)

    @pl.when((t == 0) & (b == 0))
    def _prologue():
        pltpu.make_async_copy(Wqkv_hbm.at[:, pl.ds(2 * D, D)],
                              wv_sc, sem_v).start()
        pltpu.make_async_copy(Wo_hbm, wo_sc, sem_o).start()
        pltpu.make_async_copy(W1_hbm, w1_sc, sem_1).start()
        pltpu.make_async_copy(W2_hbm, w2_sc, sem_2).start()
        pltpu.make_async_copy(Wop_hbm, wop_sc, sem_p).start()
        d2_sc[:] = jnp.zeros((1, D), jnp.float32)
        lo = jnp.dot(xlast_ref[:], Woe_ref[:],
                     preferred_element_type=jnp.float32)
        lo = lo + boe_ref[:] + pelast_ref[:]
        lolast_sc[:] = lo
        qn = _ln_rows(lo, lnqg_ref[:], lnqb_ref[:])
        qcol = jax.lax.dot_general(
            Wq_ref[:], qn, (((0,), (1,)), ((), ())),
            preferred_element_type=jnp.float32) + bqcol_ref[:]
        di = _iota((D, H), 0)
        hi = _iota((D, H), 1)
        qk0_sc[:] = jnp.dot(
            Wk_ref[:], jnp.where(di // DH == hi, qcol[:, 0:1], 0.0),
            preferred_element_type=jnp.float32)
        qk1_sc[:] = jnp.dot(
            Wk_ref[:], jnp.where(di // DH == hi, qcol[:, 1:2], 0.0),
            preferred_element_type=jnp.float32)

    @pl.when(t == 0)
    def _init_flash():
        m_sc[pl.ds(b, 1), :] = jnp.full((1, H), NEG, jnp.float32)
        l_sc[pl.ds(b, 1), :] = jnp.zeros((1, H), jnp.float32)
        acc_sc[pl.ds(b * H, H), :] = jnp.zeros((H, D), jnp.float32)

    @pl.when(b == 0)
    def _d2_accum():
        a = A_ref[:]
        d2_sc[:] = d2_sc[:] + jnp.sum(a * a, axis=0, keepdims=True)

    QK = jnp.where(b == 0, qk0_sc[:], qk1_sc[:])
    hm = jnp.where(_iota((H, D), 1) // DH == _iota((H, D), 0),
                   1.0, 0.0)

    def update(Sg, rows):
        m_old = m_sc[pl.ds(b, 1), :]
        m_new = jnp.maximum(m_old, jnp.max(Sg, axis=0, keepdims=True))
        corr = jnp.exp(m_old - m_new)
        Eu = jnp.exp(Sg - m_new)
        l_sc[pl.ds(b, 1), :] = l_sc[pl.ds(b, 1), :] * corr + jnp.sum(
            Eu, axis=0, keepdims=True)
        corr_f = jnp.dot(corr, hm, preferred_element_type=jnp.float32)
        corr_col = jax.lax.dot_general(
            hm, corr_f, (((1,), (1,)), ((), ())),
            preferred_element_type=jnp.float32) * (1.0 / DH)
        wsum = jax.lax.dot_general(
            Eu, rows, (((0,), (0,)), ((), ())),
            preferred_element_type=jnp.float32)
        acc_sc[pl.ds(b * H, H), :] = (
            acc_sc[pl.ds(b * H, H), :] * corr_col + wsum)
        m_sc[pl.ds(b, 1), :] = m_new

    x = x_ref[0]
    pe = pe_ref[:].astype(jnp.float32)
    lo = jnp.dot(x, Woe_ref[:],
                 preferred_element_type=jnp.float32) + boe_ref[:] + pe
    kvn = _ln_rows(lo, lnkg_ref[:], lnkb_ref[:])
    Sg = jnp.dot(kvn, QK, preferred_element_type=jnp.float32) * (1.0 / 8.0)
    update(Sg, kvn)

    @pl.when((t == NT - 1) & (b == 0))
    def _routing():
        state_new = _routing_compute(
            state_in_ref[:], rgw_ref[:], lnmg_ref[:], lnmb_ref[:],
            gatew_ref[:], lnsg_ref[:], lnsb_ref[:], scw_ref[:],
            d2_sc[:], read_idx_ref, experts_ref, widx_ref)
        state_sc[:] = state_new
        state_out_ref[:] = state_new

    @pl.when(t == NT - 1)
    def _state_rows():
        st = state_sc[pl.ds(b * S, S), :]
        kvn8 = _ln_rows(st, lnkg_ref[:], lnkb_ref[:])
        S8 = jnp.dot(kvn8, QK, preferred_element_type=jnp.float32) * (1.0 / 8.0)
        update(S8, kvn8)

    @pl.when((t == NT - 1) & (b == B - 1))
    def _epilogue():
        pltpu.make_async_copy(Wqkv_hbm.at[:, pl.ds(2 * D, D)],
                              wv_sc, sem_v).wait()
        pltpu.make_async_copy(Wo_hbm, wo_sc, sem_o).wait()
        pltpu.make_async_copy(W1_hbm, w1_sc, sem_1).wait()
        pltpu.make_async_copy(W2_hbm, w2_sc, sem_2).wait()
        pltpu.make_async_copy(Wop_hbm, wop_sc, sem_p).wait()
        full = jnp.dot(acc_sc[:], wv_sc[:],
                       preferred_element_type=jnp.float32)
        rows = []
        for bb in range(B):
            fb = full[bb * H:(bb + 1) * H]
            lfull = jnp.dot(l_sc[bb:bb + 1], hm,
                            preferred_element_type=jnp.float32)
            rows.append(jnp.sum(jnp.where(hm > 0.0, fb, 0.0), axis=0,
                                keepdims=True) / lfull + bv_ref[:])
        attn = jnp.concatenate(rows, axis=0)
        ao = jnp.dot(attn, wo_sc[:],
                     preferred_element_type=jnp.float32) + bo_ref[:]
        ll = lolast_sc[:] + ao
        hdn = _ln_rows(ll, lnfg_ref[:], lnfb_ref[:])
        h1 = jnp.dot(hdn, w1_sc[:],
                     preferred_element_type=jnp.float32) + b1_ref[:]
        g = 0.5 * h1 * (1.0 + jax.lax.erf(h1 * (0.5 ** 0.5)))
        ll2 = ll + jnp.dot(g, w2_sc[:],
                           preferred_element_type=jnp.float32) + b2_ref[:]
        y_ref[:] = jnp.dot(ll2, wop_sc[:],
                           preferred_element_type=jnp.float32) + bop_ref[:]


def kernel(x, state_flat, read_gate_w, ln_moe_g, ln_moe_b, gate_w, A_skew,
           ln_slot_g, ln_slot_b, slot_ctx_w, out_emb_W, out_emb_b,
           ln_q_g, ln_q_b, ln_kv_g, ln_kv_b, mha_Wqkv, mha_bqkv,
           mha_Wo, mha_bo, ln_ffn_g, ln_ffn_b, ffn_W1, ffn_b1,
           ffn_W2, ffn_b2, out_proj_W, out_proj_b):
    f32 = jnp.float32
    state16 = state_flat.reshape(B * S, D)
    row = lambda v: v.reshape(1, D)
    bq = mha_bqkv[:D]
    bv = mha_bqkv[2 * D:]

    cst = lambda shape: pl.BlockSpec(shape, lambda t, b: tuple(
        0 for _ in shape))
    hbm = pl.BlockSpec(memory_space=pltpu.MemorySpace.HBM)

    y, read_idx, experts, write_idx, state_new = pl.pallas_call(
        _mega_kernel,
        grid=(NT, B),
        in_specs=[
            pl.BlockSpec((1, TK, D), lambda t, b: (b, t, 0)),
            pl.BlockSpec((TK, D), lambda t, b: (t, 0)),
            pl.BlockSpec((TA, D), lambda t, b: (t, 0)),
            cst((D, D)),
            cst((1, D)),
            cst((1, D)), cst((1, D)),
            cst((B, D)),
            cst((1, D)),
            cst((1, D)), cst((1, D)),
            pl.BlockSpec((D, D), lambda t, b: (0, 0)),
            cst((D, 1)),
            pl.BlockSpec((D, D), lambda t, b: (0, 1)),
            cst((B * S, D)),
            cst((1, D)),
            cst((1, D)), cst((1, D)),
            cst((D, E)),
            cst((1, D)), cst((1, D)),
            cst((1, D)),
            hbm, hbm, hbm, hbm, hbm,
            cst((1, D)), cst((1, D)),
            cst((1, D)), cst((1, D)),
            cst((1, D)), cst((1, D)),
            cst((1, D)),
        ],
        out_specs=(
            cst((B, D)),
            cst((B, KR)),
            cst((B * KR, KT)),
            cst((B, KW)),
            cst((B * S, D)),
        ),
        out_shape=(
            jax.ShapeDtypeStruct((B, D), f32),
            jax.ShapeDtypeStruct((B, KR), jnp.int32),
            jax.ShapeDtypeStruct((B * KR, KT), jnp.int32),
            jax.ShapeDtypeStruct((B, KW), jnp.int32),
            jax.ShapeDtypeStruct((B * S, D), f32),
        ),
        scratch_shapes=[
            pltpu.VMEM((D, H), f32), pltpu.VMEM((D, H), f32),
            pltpu.VMEM((B, D), f32),
            pltpu.VMEM((B, H), f32), pltpu.VMEM((B, H), f32),
            pltpu.VMEM((B * H, D), f32),
            pltpu.VMEM((1, D), f32),
            pltpu.VMEM((B * S, D), f32),
            pltpu.VMEM((D, D), f32), pltpu.VMEM((D, D), f32),
            pltpu.VMEM((D, D), f32), pltpu.VMEM((D, D), f32),
            pltpu.VMEM((D, D), f32),
            pltpu.SemaphoreType.DMA, pltpu.SemaphoreType.DMA,
            pltpu.SemaphoreType.DMA, pltpu.SemaphoreType.DMA,
            pltpu.SemaphoreType.DMA,
        ],
    )(x, _PE_BF, A_skew, out_emb_W, row(out_emb_b), row(ln_kv_g),
      row(ln_kv_b),
      x[:, -1, :], _PE_LAST, row(ln_q_g), row(ln_q_b), mha_Wqkv,
      bq.reshape(D, 1), mha_Wqkv,
      state16, row(read_gate_w), row(ln_moe_g), row(ln_moe_b), gate_w,
      row(ln_slot_g), row(ln_slot_b), row(slot_ctx_w),
      mha_Wqkv, mha_Wo, ffn_W1, ffn_W2, out_proj_W,
      bv.reshape(1, D), row(mha_bo), row(ln_ffn_g), row(ln_ffn_b),
      row(ffn_b1), row(ffn_b2), row(out_proj_b))

    return (y, experts.reshape(B, KR, KT), read_idx, write_idx,
            state_new.reshape(B, S * D))
